# baseline (speedup 1.0000x reference)
import jax
import jax.numpy as jnp
from jax import lax
from jax.experimental import pallas as pl
from jax.experimental.pallas import tpu as pltpu
from jax.experimental.pallas import tpu_sc as plsc

NUM_CORES = 2
NUM_SUBCORES = 16
NUM_WORKERS = NUM_CORES * NUM_SUBCORES
WIN = 128
NBUF = 2
IBUF = 4
LANES = 16
DUMMY_WIN = 192


def _round_up(v, m):
    return (v + m - 1) // m * m


def _sc_aggregate(x, row2d, col2d, n_pad, nwin):
    d = x.shape[1]
    nw = nwin // NUM_SUBCORES
    assert nw % IBUF == 0 and nw >= 2 * IBUF
    assert DUMMY_WIN % IBUF == 0 and 2 * IBUF <= DUMMY_WIN <= nwin
    zrows = n_pad // NUM_SUBCORES
    assert zrows % WIN == 0
    mesh = plsc.VectorSubcoreMesh(core_axis_name="c", subcore_axis_name="s")

    def body(x_hbm, row_hbm, col_hbm, out_hbm,
             cidx, ridx, rows0, rows1, acc,
             is0, is1, is2, is3, gs0, gs1):
        bufs = (rows0, rows1)
        gsems = (gs0, gs1)
        isems = (is0, is1, is2, is3)
        c = lax.axis_index("c")
        s = lax.axis_index("s")
        mywin = jnp.where(c == 0, nw, DUMMY_WIN)
        base = jnp.where(c == 0, s * nw, 0)

        def idx_issue(u, k):
            pltpu.async_copy(row_hbm.at[base + u], ridx.at[k], isems[k])
            pltpu.async_copy(col_hbm.at[base + u], cidx.at[k], isems[k])

        def idx_wait(k):
            pltpu.make_async_copy(row_hbm.at[0], ridx.at[k], isems[k]).wait()
            pltpu.make_async_copy(col_hbm.at[0], cidx.at[k], isems[k]).wait()

        def gather_issue(u, k, b):
            idx_wait(k)
            pltpu.async_copy(x_hbm.at[cidx.at[k]], bufs[b], gsems[b])

        def gather_wait(k, b):
            pltpu.make_async_copy(x_hbm.at[cidx.at[k]],
                                  bufs[b], gsems[b]).wait()

        @pl.when(c == 0)
        def _():
            @pl.loop(0, WIN)
            def _(r):
                @pl.loop(0, d // LANES)
                def _(k):
                    rows0[r, pl.ds(k * LANES, LANES)] = jnp.zeros(
                        (LANES,), jnp.float32)

            @pl.loop(0, zrows // WIN)
            def _(b):
                pltpu.sync_copy(rows0,
                                acc.at[pl.ds(s * zrows + b * WIN, WIN)])

            plsc.subcore_barrier()

        def window(u, k, b, issue_idx, issue_gather):
            gather_wait(k, b)
            pltpu.sync_copy(bufs[b], acc.at[ridx.at[k]], add=True)
            if issue_idx:
                idx_issue(u + IBUF, k)
            if issue_gather:
                gather_issue(u + NBUF, (k + NBUF) % IBUF, b)

        for k in range(IBUF):
            idx_issue(k, k)
        gather_issue(0, 0, 0)
        gather_issue(1, 1, 1)

        @pl.loop(0, mywin - 2 * IBUF, step=IBUF)
        def _(t):
            for k in range(IBUF):
                window(t + k, k, k % NBUF, True, True)

        t0 = mywin - 2 * IBUF
        for k in range(IBUF):
            window(t0 + k, k, k % NBUF, True, True)
        for k in range(IBUF):
            window(t0 + IBUF + k, k, k % NBUF, False, k < IBUF - NBUF)

        @pl.when(c == 0)
        def _():
            plsc.subcore_barrier()
            pltpu.sync_copy(acc.at[pl.ds(s * zrows, zrows)],
                            out_hbm.at[0, pl.ds(s * zrows, zrows)])

    kern = pl.kernel(
        body,
        out_type=jax.ShapeDtypeStruct((1, n_pad, d), jnp.float32),
        mesh=mesh,
        scratch_types=[
            pltpu.VMEM((IBUF, WIN), jnp.int32),
            pltpu.VMEM((IBUF, WIN), jnp.int32),
            pltpu.VMEM((WIN, d), jnp.float32),
            pltpu.VMEM((WIN, d), jnp.float32),
            pltpu.VMEM_SHARED((n_pad, d), jnp.float32),
            pltpu.SemaphoreType.DMA,
            pltpu.SemaphoreType.DMA,
            pltpu.SemaphoreType.DMA,
            pltpu.SemaphoreType.DMA,
            pltpu.SemaphoreType.DMA,
            pltpu.SemaphoreType.DMA,
        ],
    )
    return kern(x, row2d, col2d)


def _tc_combine(p, x, w_mat):
    n, d = x.shape
    br = 1000
    assert n % br == 0

    def body(p_ref, x_ref, w_ref, o_ref):
        agg = p_ref[0] + x_ref[...]
        o_ref[...] = jnp.dot(agg, w_ref[...],
                             preferred_element_type=jnp.float32)

    return pl.pallas_call(
        body,
        grid=(n // br,),
        in_specs=[
            pl.BlockSpec((1, br, d), lambda i: (0, i, 0)),
            pl.BlockSpec((br, d), lambda i: (i, 0)),
            pl.BlockSpec((d, d), lambda i: (0, 0)),
        ],
        out_specs=pl.BlockSpec((br, d), lambda i: (i, 0)),
        out_shape=jax.ShapeDtypeStruct((n, d), jnp.float32),
    )(p, x, w_mat)


@jax.jit
def kernel(x, edge_index, W):
    n, d = x.shape
    e = edge_index.shape[1]
    e_pad = _round_up(e, NUM_SUBCORES * WIN * IBUF)
    nwin = e_pad // WIN
    n_pad = _round_up(n + WIN, NUM_SUBCORES * WIN)

    pad = e_pad - e
    trash = n + (jnp.arange(pad, dtype=edge_index.dtype) % WIN)
    row = jnp.concatenate([edge_index[0], trash])
    col = jnp.concatenate(
        [edge_index[1], jnp.zeros((pad,), edge_index.dtype)])

    p = _sc_aggregate(x, row.reshape(-1, WIN), col.reshape(-1, WIN),
                      n_pad, nwin)
    return _tc_combine(p, x, W)


# speedup vs baseline: 4.0020x; 4.0020x over previous
import jax
import jax.numpy as jnp
from jax import lax
from jax.experimental import pallas as pl
from jax.experimental.pallas import tpu as pltpu
from jax.experimental.pallas import tpu_sc as plsc

NUM_CORES = 2
NUM_SUBCORES = 16
NUM_WORKERS = NUM_CORES * NUM_SUBCORES
WIN = 128
NBUF = 2
IBUF = 4
LANES = 16


def _round_up(v, m):
    return (v + m - 1) // m * m


def _sc_aggregate(x, row2d, col2d, n_pad, nwin):
    d = x.shape[1]
    nw = nwin // NUM_WORKERS
    assert nw % IBUF == 0 and nw >= 2 * IBUF
    zrows = n_pad // NUM_SUBCORES
    assert zrows % WIN == 0
    mesh = plsc.VectorSubcoreMesh(core_axis_name="c", subcore_axis_name="s")

    def body(x_hbm, row_hbm, col_hbm, out_hbm,
             cidx, ridx, rows0, rows1, acc,
             is0, is1, is2, is3, gs0, gs1):
        bufs = (rows0, rows1)
        gsems = (gs0, gs1)
        isems = (is0, is1, is2, is3)
        c = lax.axis_index("c")
        s = lax.axis_index("s")
        base = (s * NUM_CORES + c) * nw

        def idx_issue(u, k):
            pltpu.async_copy(row_hbm.at[base + u], ridx.at[k], isems[k])
            pltpu.async_copy(col_hbm.at[base + u], cidx.at[k], isems[k])

        def idx_wait(k):
            pltpu.make_async_copy(row_hbm.at[0], ridx.at[k], isems[k]).wait()
            pltpu.make_async_copy(col_hbm.at[0], cidx.at[k], isems[k]).wait()

        def gather_issue(u, k, b):
            idx_wait(k)
            pltpu.async_copy(x_hbm.at[cidx.at[k]], bufs[b], gsems[b])

        def gather_wait(k, b):
            pltpu.make_async_copy(x_hbm.at[cidx.at[k]],
                                  bufs[b], gsems[b]).wait()

        @pl.loop(0, WIN)
        def _(r):
            @pl.loop(0, d // LANES)
            def _(k):
                rows0[r, pl.ds(k * LANES, LANES)] = jnp.zeros(
                    (LANES,), jnp.float32)

        @pl.loop(0, zrows // WIN)
        def _(b):
            pltpu.sync_copy(rows0,
                            acc.at[pl.ds(s * zrows + b * WIN, WIN)])

        plsc.subcore_barrier()

        def window(u, k, b, issue_idx, issue_gather):
            gather_wait(k, b)
            pltpu.sync_copy(bufs[b], acc.at[ridx.at[k]], add=True)
            if issue_idx:
                idx_issue(u + IBUF, k)
            if issue_gather:
                gather_issue(u + NBUF, (k + NBUF) % IBUF, b)

        for k in range(IBUF):
            idx_issue(k, k)
        gather_issue(0, 0, 0)
        gather_issue(1, 1, 1)

        @pl.loop(0, nw - 2 * IBUF, step=IBUF)
        def _(t):
            for k in range(IBUF):
                window(t + k, k, k % NBUF, True, True)

        t0 = nw - 2 * IBUF
        for k in range(IBUF):
            window(t0 + k, k, k % NBUF, True, True)
        for k in range(IBUF):
            window(t0 + IBUF + k, k, k % NBUF, False, k < IBUF - NBUF)

        plsc.subcore_barrier()
        pltpu.sync_copy(acc.at[pl.ds(s * zrows, zrows)],
                        out_hbm.at[c, pl.ds(s * zrows, zrows)])

    kern = pl.kernel(
        body,
        out_type=jax.ShapeDtypeStruct((NUM_CORES, n_pad, d), jnp.float32),
        mesh=mesh,
        scratch_types=[
            pltpu.VMEM((IBUF, WIN), jnp.int32),
            pltpu.VMEM((IBUF, WIN), jnp.int32),
            pltpu.VMEM((WIN, d), jnp.float32),
            pltpu.VMEM((WIN, d), jnp.float32),
            pltpu.VMEM_SHARED((n_pad, d), jnp.float32),
            pltpu.SemaphoreType.DMA,
            pltpu.SemaphoreType.DMA,
            pltpu.SemaphoreType.DMA,
            pltpu.SemaphoreType.DMA,
            pltpu.SemaphoreType.DMA,
            pltpu.SemaphoreType.DMA,
        ],
    )
    return kern(x, row2d, col2d)


def _tc_combine(p, x, w_mat):
    n, d = x.shape
    br = 1000
    assert n % br == 0

    def body(p0_ref, p1_ref, x_ref, w_ref, o_ref):
        agg = p0_ref[0] + p1_ref[0] + x_ref[...]
        o_ref[...] = jnp.dot(agg, w_ref[...],
                             preferred_element_type=jnp.float32)

    return pl.pallas_call(
        body,
        grid=(n // br,),
        in_specs=[
            pl.BlockSpec((1, br, d), lambda i: (0, i, 0)),
            pl.BlockSpec((1, br, d), lambda i: (1, i, 0)),
            pl.BlockSpec((br, d), lambda i: (i, 0)),
            pl.BlockSpec((d, d), lambda i: (0, 0)),
        ],
        out_specs=pl.BlockSpec((br, d), lambda i: (i, 0)),
        out_shape=jax.ShapeDtypeStruct((n, d), jnp.float32),
    )(p, p, x, w_mat)


@jax.jit
def kernel(x, edge_index, W):
    n, d = x.shape
    e = edge_index.shape[1]
    e_pad = _round_up(e, NUM_WORKERS * WIN * IBUF)
    nwin = e_pad // WIN
    n_pad = _round_up(n + WIN, NUM_SUBCORES * WIN)

    pad = e_pad - e
    ar = jnp.arange(pad, dtype=edge_index.dtype)
    row = jnp.concatenate([edge_index[0], n + ar % WIN])
    col = jnp.concatenate([edge_index[1], ar % n])

    p = _sc_aggregate(x, row.reshape(-1, WIN), col.reshape(-1, WIN),
                      n_pad, nwin)
    return _tc_combine(p, x, W)
